# baseline (speedup 1.0000x reference)
import functools

import jax
import jax.numpy as jnp
from jax import lax
from jax.experimental import pallas as pl
from jax.experimental.pallas import tpu as pltpu
from jax.experimental.pallas import tpu_sc as plsc

_BLOCK_SIZE = 8192
_N_EMBD = 1024

_info = plsc.get_sparse_core_info()
_NC, _NS = _info.num_cores, _info.num_subcores
_NW = _NC * _NS
_ROWS_PER_W = _BLOCK_SIZE // _NW

_CHUNK = 16
_NBUF = 6
_NCHUNKS = _ROWS_PER_W // _CHUNK


def _sc_copy(wpe_hbm, out_hbm, buf, *sems):
    sin = sems[:_NBUF]
    sout = sems[_NBUF:]
    wid = lax.axis_index("s") * _NC + lax.axis_index("c")
    base = wid * _ROWS_PER_W

    def cin(i):
        b = i % _NBUF
        return pltpu.async_copy(
            wpe_hbm.at[pl.ds(base + i * _CHUNK, _CHUNK)], buf.at[b], sin[b]
        )

    def cout(i):
        b = i % _NBUF
        return pltpu.async_copy(
            buf.at[b], out_hbm.at[pl.ds(base + i * _CHUNK, _CHUNK)], sout[b]
        )

    ins = [None] * _NCHUNKS
    outs = [None] * _NCHUNKS
    for i in range(_NCHUNKS):
        if i >= _NBUF:
            outs[i - _NBUF].wait()
        ins[i] = cin(i)
        if i >= 1:
            ins[i - 1].wait()
            outs[i - 1] = cout(i - 1)
    ins[_NCHUNKS - 1].wait()
    outs[_NCHUNKS - 1] = cout(_NCHUNKS - 1)
    for j in range(_NCHUNKS - _NBUF, _NCHUNKS):
        outs[j].wait()


def kernel(wpe):
    mesh = plsc.VectorSubcoreMesh(core_axis_name="c", subcore_axis_name="s")
    run = functools.partial(
        pl.kernel,
        mesh=mesh,
        out_type=jax.ShapeDtypeStruct((_BLOCK_SIZE, _N_EMBD), jnp.float32),
        scratch_types=(
            [pltpu.VMEM((_NBUF, _CHUNK, _N_EMBD), jnp.float32)]
            + [pltpu.SemaphoreType.DMA] * (2 * _NBUF)
        ),
    )(_sc_copy)
    return run(wpe)[None]


# speedup vs baseline: 1.0113x; 1.0113x over previous
import functools

import jax
import jax.numpy as jnp
from jax import lax
from jax.experimental import pallas as pl
from jax.experimental.pallas import tpu as pltpu
from jax.experimental.pallas import tpu_sc as plsc

_BLOCK_SIZE = 8192
_N_EMBD = 1024

_info = plsc.get_sparse_core_info()
_NC, _NS = _info.num_cores, _info.num_subcores
_NW = _NC * _NS
_ROWS_PER_W = _BLOCK_SIZE // _NW

_CHUNK = 32
_NBUF = 2
_NCHUNKS = _ROWS_PER_W // _CHUNK


def _sc_copy(wpe_hbm, out_hbm, buf, *sems):
    sin = sems[:_NBUF]
    sout = sems[_NBUF:]
    wid = lax.axis_index("s") * _NC + lax.axis_index("c")
    base = wid * _ROWS_PER_W

    def cin(i):
        b = i % _NBUF
        return pltpu.async_copy(
            wpe_hbm.at[pl.ds(base + i * _CHUNK, _CHUNK)], buf.at[b], sin[b]
        )

    def cout(i):
        b = i % _NBUF
        return pltpu.async_copy(
            buf.at[b], out_hbm.at[pl.ds(base + i * _CHUNK, _CHUNK)], sout[b]
        )

    ins = [None] * _NCHUNKS
    outs = [None] * _NCHUNKS
    for i in range(_NCHUNKS):
        if i >= _NBUF:
            outs[i - _NBUF].wait()
        ins[i] = cin(i)
        if i >= 1:
            ins[i - 1].wait()
            outs[i - 1] = cout(i - 1)
    ins[_NCHUNKS - 1].wait()
    outs[_NCHUNKS - 1] = cout(_NCHUNKS - 1)
    for j in range(_NCHUNKS - _NBUF, _NCHUNKS):
        outs[j].wait()


def kernel(wpe):
    mesh = plsc.VectorSubcoreMesh(core_axis_name="c", subcore_axis_name="s")
    run = functools.partial(
        pl.kernel,
        mesh=mesh,
        out_type=jax.ShapeDtypeStruct((_BLOCK_SIZE, _N_EMBD), jnp.float32),
        scratch_types=(
            [pltpu.VMEM((_NBUF, _CHUNK, _N_EMBD), jnp.float32)]
            + [pltpu.SemaphoreType.DMA] * (2 * _NBUF)
        ),
    )(_sc_copy)
    return run(wpe)[None]


# speedup vs baseline: 1.0117x; 1.0004x over previous
import functools

import jax
import jax.numpy as jnp
from jax import lax
from jax.experimental import pallas as pl
from jax.experimental.pallas import tpu as pltpu
from jax.experimental.pallas import tpu_sc as plsc

_BLOCK_SIZE = 8192
_N_EMBD = 1024

_info = plsc.get_sparse_core_info()
_NC, _NS = _info.num_cores, _info.num_subcores
_NW = _NC * _NS
_ROWS_PER_W = _BLOCK_SIZE // _NW

_CHUNK = 32
_NBUF = 2
_NCHUNKS = _ROWS_PER_W // _CHUNK


def _sc_copy(wpe_hbm, out_hbm, buf, *sems):
    sin = sems[:_NBUF]
    sout = sems[_NBUF:]
    wid = lax.axis_index("c") * _NS + lax.axis_index("s")
    base = wid * _ROWS_PER_W

    def cin(i):
        b = i % _NBUF
        return pltpu.async_copy(
            wpe_hbm.at[pl.ds(base + i * _CHUNK, _CHUNK)], buf.at[b], sin[b]
        )

    def cout(i):
        b = i % _NBUF
        return pltpu.async_copy(
            buf.at[b], out_hbm.at[pl.ds(base + i * _CHUNK, _CHUNK)], sout[b]
        )

    ins = [None] * _NCHUNKS
    outs = [None] * _NCHUNKS
    for i in range(_NCHUNKS):
        if i >= _NBUF:
            outs[i - _NBUF].wait()
        ins[i] = cin(i)
        if i >= 1:
            ins[i - 1].wait()
            outs[i - 1] = cout(i - 1)
    ins[_NCHUNKS - 1].wait()
    outs[_NCHUNKS - 1] = cout(_NCHUNKS - 1)
    for j in range(_NCHUNKS - _NBUF, _NCHUNKS):
        outs[j].wait()


def kernel(wpe):
    mesh = plsc.VectorSubcoreMesh(core_axis_name="c", subcore_axis_name="s")
    run = functools.partial(
        pl.kernel,
        mesh=mesh,
        out_type=jax.ShapeDtypeStruct((_BLOCK_SIZE, _N_EMBD), jnp.float32),
        scratch_types=(
            [pltpu.VMEM((_NBUF, _CHUNK, _N_EMBD), jnp.float32)]
            + [pltpu.SemaphoreType.DMA] * (2 * _NBUF)
        ),
    )(_sc_copy)
    return run(wpe)[None]
